# Initial kernel scaffold; baseline (speedup 1.0000x reference)
import jax
import jax.numpy as jnp
from jax.experimental import pallas as pl


def kernel(x, edge_index, W1, b1, W2, b2, gamma, W_proj, b_proj):
    raise NotImplementedError("write your pallas kernel here")


# speedup vs baseline: 44.0194x; 44.0194x over previous
import functools

import jax
import jax.numpy as jnp
from jax import lax
from jax.experimental import pallas as pl
from jax.experimental.pallas import tpu as pltpu
from jax.experimental.pallas import tpu_sc as plsc

_N = 100000
_E = 3200000
_D = 128
_H = 64
_C = 16
_K = 10

_NC = 2
_NS = 16
_NW = _NC * _NS
_EPW = _E // _NW
_CH = 800
_NCH = _EPW // _CH
_NPAD = 100096
_SPAN = _NPAD // _NS
_NZC = _SPAN // _CH
_REM = _SPAN - _NZC * _CH

_NP8 = _N // 8
_SROW = 2 * _NPAD * _C // 128
_SROW1 = _SROW // 2
_BP = 256
_GRIDP = (_NP8 + _BP - 1) // _BP

_sc_mesh = plsc.VectorSubcoreMesh(core_axis_name="c", subcore_axis_name="s")


def _deg_prep_body(src_h, dst_h, dstp_h, degp_h,
                   src_v, dst_v, srcp_v, dstp_v, ones_v, zero_v, acc, sem):
    cid = lax.axis_index("c")
    sid = lax.axis_index("s")
    wid = sid * _NC + cid

    @pl.loop(0, _CH // 16)
    def _fill(i):
        ones_v[pl.ds(i * 16, 16)] = jnp.full((16,), 1.0, jnp.float32)
        zero_v[pl.ds(i * 16, 16)] = jnp.zeros((16,), jnp.float32)

    zb = sid * _SPAN
    for j in range(_NZC):
        pltpu.sync_copy(zero_v, acc.at[pl.ds(zb + j * _CH, _CH)])
    pltpu.sync_copy(zero_v.at[pl.ds(0, _REM)],
                    acc.at[pl.ds(zb + _NZC * _CH, _REM)])
    plsc.subcore_barrier()

    ebase = wid * _EPW

    @pl.loop(0, _NCH)
    def _chunk(c):
        base = pl.multiple_of(ebase + c * _CH, 8)
        pltpu.sync_copy(src_h.at[pl.ds(base, _CH)], src_v)
        pltpu.sync_copy(dst_h.at[pl.ds(base, _CH)], dst_v)

        @pl.loop(0, _CH // 16)
        def _vec(i):
            s = src_v[pl.ds(i * 16, 16)]
            d = dst_v[pl.ds(i * 16, 16)]
            eq = s == d
            srcp_v[pl.ds(i * 16, 16)] = jnp.where(eq, _N, s)
            dstp_v[pl.ds(i * 16, 16)] = jnp.where(eq, _N, d)

        pltpu.sync_copy(ones_v, acc.at[srcp_v], add=True)
        pltpu.sync_copy(dstp_v, dstp_h.at[pl.ds(base, _CH)])

    plsc.subcore_barrier()
    ob = cid * _NPAD + zb
    for j in range(_NZC):
        pltpu.sync_copy(acc.at[pl.ds(zb + j * _CH, _CH)], zero_v)
        pltpu.sync_copy(zero_v, degp_h.at[pl.ds(ob + j * _CH, _CH)])
    pltpu.sync_copy(acc.at[pl.ds(zb + _NZC * _CH, _REM)],
                    zero_v.at[pl.ds(0, _REM)])
    pltpu.sync_copy(zero_v.at[pl.ds(0, _REM)],
                    degp_h.at[pl.ds(ob + _NZC * _CH, _REM)])


@functools.partial(
    pl.kernel,
    out_type=(jax.ShapeDtypeStruct((_E,), jnp.int32),
              jax.ShapeDtypeStruct((2 * _NPAD,), jnp.float32)),
    mesh=_sc_mesh,
    scratch_types=[
        pltpu.VMEM((_CH,), jnp.int32),
        pltpu.VMEM((_CH,), jnp.int32),
        pltpu.VMEM((_CH,), jnp.int32),
        pltpu.VMEM((_CH,), jnp.int32),
        pltpu.VMEM((_CH,), jnp.float32),
        pltpu.VMEM((_CH,), jnp.float32),
        pltpu.VMEM_SHARED((_NPAD,), jnp.float32),
        pltpu.SemaphoreType.DMA,
    ],
    compiler_params=pltpu.CompilerParams(use_tc_tiling_on_sc=False),
)
def _deg_prep(*args):
    _deg_prep_body(*args)


def _prop_body(u_h, src_h, dstp_h, s_h, sidx, didx, rows, acc, sem):
    cid = lax.axis_index("c")
    sid = lax.axis_index("s")
    wid = sid * _NC + cid

    @pl.loop(0, _CH)
    def _z(i):
        rows[i, :] = jnp.zeros((16,), jnp.float32)

    zb = sid * _SPAN
    for j in range(_NZC):
        pltpu.sync_copy(rows, acc.at[pl.ds(zb + j * _CH, _CH)])
    pltpu.sync_copy(rows.at[pl.ds(0, _REM)],
                    acc.at[pl.ds(zb + _NZC * _CH, _REM)])
    plsc.subcore_barrier()

    ebase = wid * _EPW

    @pl.loop(0, _NCH)
    def _chunk(c):
        base = pl.multiple_of(ebase + c * _CH, 8)
        pltpu.sync_copy(src_h.at[pl.ds(base, _CH)], sidx)
        pltpu.sync_copy(dstp_h.at[pl.ds(base, _CH)], didx)
        pltpu.async_copy(u_h.at[sidx], rows, sem).wait()
        pltpu.sync_copy(rows, acc.at[didx], add=True)

    plsc.subcore_barrier()
    ob = cid * _NPAD + zb
    for j in range(_NZC):
        pltpu.sync_copy(acc.at[pl.ds(zb + j * _CH, _CH)], rows)
        pltpu.sync_copy(rows, s_h.at[pl.ds(ob + j * _CH, _CH)])
    pltpu.sync_copy(acc.at[pl.ds(zb + _NZC * _CH, _REM)],
                    rows.at[pl.ds(0, _REM)])
    pltpu.sync_copy(rows.at[pl.ds(0, _REM)],
                    s_h.at[pl.ds(ob + _NZC * _CH, _REM)])


@functools.partial(
    pl.kernel,
    out_type=jax.ShapeDtypeStruct((2 * _NPAD, _C), jnp.float32),
    mesh=_sc_mesh,
    scratch_types=[
        pltpu.VMEM((_CH,), jnp.int32),
        pltpu.VMEM((_CH,), jnp.int32),
        pltpu.VMEM((_CH, _C), jnp.float32),
        pltpu.VMEM_SHARED((_NPAD, _C), jnp.float32),
        pltpu.SemaphoreType.DMA,
    ],
    compiler_params=pltpu.CompilerParams(use_tc_tiling_on_sc=False),
)
def _prop(*args):
    _prop_body(*args)


def _e8(dtype=jnp.float32):
    grp = lax.broadcasted_iota(jnp.int32, (8, 128), 1) // 16
    row = lax.broadcasted_iota(jnp.int32, (8, 128), 0)
    return (grp == row).astype(dtype)


def _e8t(dtype=jnp.float32):
    grp = lax.broadcasted_iota(jnp.int32, (128, 8), 0) // 16
    col = lax.broadcasted_iota(jnp.int32, (128, 8), 1)
    return (grp == col).astype(dtype)


_ROWB = pl.BlockSpec((_BP, 128), lambda i: (i, 0))
_S2B = pl.BlockSpec((2, _BP, 128), lambda i: (0, i, 0))


def _mlp_prep_body(xp_ref, w1b_ref, b1b_ref, w2b_ref, b2b_ref, deg_ref,
                   h_ref, disb_ref, d2b_ref, u0_ref):
    h1 = lax.dot_general(xp_ref[...], w1b_ref[...], (((1,), (0,)), ((), ())),
                         preferred_element_type=jnp.float32)
    h1 = jnp.maximum(h1 + b1b_ref[...], 0.0)
    h = lax.dot_general(h1, w2b_ref[...], (((1,), (0,)), ((), ())),
                        preferred_element_type=jnp.float32) + b2b_ref[...]
    deg8 = deg_ref[0] + deg_ref[1]
    dis8 = jnp.where(deg8 > 0.0, lax.rsqrt(deg8), 0.0)
    disb = lax.dot_general(dis8, _e8(), (((1,), (0,)), ((), ())),
                           preferred_element_type=jnp.float32)
    h_ref[...] = h
    disb_ref[...] = disb
    d2b_ref[...] = disb * disb
    u0_ref[...] = disb * h


def _mlp_prep(xp, W1B, b1B, W2B, b2B, deg3):
    return pl.pallas_call(
        _mlp_prep_body,
        grid=(_GRIDP,),
        in_specs=[
            pl.BlockSpec((_BP, 8 * _D), lambda i: (i, 0)),
            pl.BlockSpec((8 * _D, 8 * _H), lambda i: (0, 0)),
            pl.BlockSpec((1, 8 * _H), lambda i: (0, 0)),
            pl.BlockSpec((8 * _H, 128), lambda i: (0, 0)),
            pl.BlockSpec((1, 128), lambda i: (0, 0)),
            pl.BlockSpec((2, _BP, 8), lambda i: (0, i, 0)),
        ],
        out_specs=[_ROWB] * 4,
        out_shape=[jax.ShapeDtypeStruct((_NP8, 128), jnp.float32)] * 4,
    )(xp, W1B, b1B, W2B, b2B, deg3)


def _unext_first_body(s_ref, d2_ref, u_ref):
    u_ref[...] = -d2_ref[...] * (s_ref[0] + s_ref[1])


def _unext_mid_body(s_ref, d2_ref, up_ref, u_ref):
    u_ref[...] = -2.0 * d2_ref[...] * (s_ref[0] + s_ref[1]) - up_ref[...]


def _unext_first(S2, d2b):
    return pl.pallas_call(
        _unext_first_body, grid=(_GRIDP,),
        in_specs=[_S2B, _ROWB], out_specs=_ROWB,
        out_shape=jax.ShapeDtypeStruct((_NP8, 128), jnp.float32),
    )(S2, d2b)


def _unext_mid(S2, d2b, uprev):
    return pl.pallas_call(
        _unext_mid_body, grid=(_GRIDP,),
        in_specs=[_S2B, _ROWB, _ROWB], out_specs=_ROWB,
        out_shape=jax.ShapeDtypeStruct((_NP8, 128), jnp.float32),
    )(S2, d2b, uprev)


def _final_body(*refs):
    s_refs = refs[:_K]
    h_ref, disb_ref, wp_ref, bp_ref, g_ref, out_ref = refs[_K:]
    e8 = _e8()
    e8t = _e8t()
    wp = wp_ref[...]
    bp = bp_ref[0, 0]
    disb = disb_ref[...]

    def term(tx, k):
        r8 = lax.dot_general(tx * wp, e8t, (((1,), (0,)), ((), ())),
                             preferred_element_type=jnp.float32) + bp
        e = g_ref[0, k] / (1.0 + jnp.exp(-r8))
        eta = lax.dot_general(e, e8, (((1,), (0,)), ((), ())),
                              preferred_element_type=jnp.float32)
        return eta * tx

    tx_a = h_ref[...]
    acc = term(tx_a, 0)
    tx_b = -disb * (s_refs[0][0] + s_refs[0][1])
    acc += term(tx_b, 1)
    for k in range(2, _K + 1):
        s = s_refs[k - 1][0] + s_refs[k - 1][1]
        tx_n = -2.0 * disb * s - tx_a
        acc += term(tx_n, k)
        tx_a, tx_b = tx_b, tx_n

    m = jnp.max(acc, axis=1, keepdims=True)
    z = acc - m
    s8 = lax.dot_general(jnp.exp(z), e8t, (((1,), (0,)), ((), ())),
                         preferred_element_type=jnp.float32)
    l128 = lax.dot_general(jnp.log(s8), e8, (((1,), (0,)), ((), ())),
                           preferred_element_type=jnp.float32)
    out_ref[...] = z - l128


def _final(Sp, h, disb, wp128, bpr, gamma):
    return pl.pallas_call(
        _final_body,
        grid=(_GRIDP,),
        in_specs=[_S2B] * _K + [
            _ROWB, _ROWB,
            pl.BlockSpec((1, 128), lambda i: (0, 0)),
            pl.BlockSpec(memory_space=pltpu.SMEM),
            pl.BlockSpec(memory_space=pltpu.SMEM),
        ],
        out_specs=_ROWB,
        out_shape=jax.ShapeDtypeStruct((_NP8, 128), jnp.float32),
    )(*Sp, h, disb, wp128, bpr, gamma)


def kernel(x, edge_index, W1, b1, W2, b2, gamma, W_proj, b_proj):
    src = edge_index[0]
    dst = edge_index[1]
    eye8 = jnp.eye(8, dtype=jnp.float32)
    W1B = jnp.kron(eye8, W1.T)
    W2B = jnp.kron(eye8, W2.T)
    b1B = jnp.tile(b1.reshape(1, _H), (1, 8))
    b2B = jnp.tile(b2.reshape(1, _C), (1, 8))
    wp128 = jnp.tile(W_proj.reshape(1, _C), (1, 8))
    bpr = b_proj.reshape(1, 1)
    xp = x.reshape(_NP8, 8 * _D)

    dstp, degp = _deg_prep(src, dst)
    deg3 = degp.reshape(2, _NPAD)[:, :_N].reshape(2, _NP8, 8)
    h, disb, d2b, u0 = _mlp_prep(xp, W1B, b1B, W2B, b2B, deg3)

    Sp = []
    upk = u0
    S0 = _prop(u0.reshape(_N, _C), src, dstp).reshape(2, _SROW1, 128)
    Sp.append(S0)
    uk = _unext_first(S0, d2b)
    for k in range(1, _K):
        Sk = _prop(uk.reshape(_N, _C), src, dstp).reshape(2, _SROW1, 128)
        Sp.append(Sk)
        if k < _K - 1:
            uk, upk = _unext_mid(Sk, d2b, upk), uk

    out = _final(Sp, h, disb, wp128, bpr, gamma)
    return out.reshape(_N, _C)


# speedup vs baseline: 68.7284x; 1.5613x over previous
import functools

import jax
import jax.numpy as jnp
from jax import lax
from jax.experimental import pallas as pl
from jax.experimental.pallas import tpu as pltpu
from jax.experimental.pallas import tpu_sc as plsc

_N = 100000
_E = 3200000
_D = 128
_H = 64
_C = 16
_K = 10

_NC = 2
_NS = 16
_NW = _NC * _NS
_EPW = _E // _NW
_CH = 800
_NCH = _EPW // _CH
_NPAD = 100096
_SPAN = _NPAD // _NS
_NZC = _SPAN // _CH
_REM = _SPAN - _NZC * _CH

_RW = 400
_NROW = _E // _RW
_NSEC = _EPW // _CH

_NP8 = _N // 8
_SROW = 2 * _NPAD * _C // 128
_SROW1 = _SROW // 2
_BP = 256
_GRIDP = (_NP8 + _BP - 1) // _BP

_sc_mesh = plsc.VectorSubcoreMesh(core_axis_name="c", subcore_axis_name="s")


def _deg_prep_body(src_h, dst_h, dstp_h, degp_h,
                   src_v, dst_v, srcp_v, dstp_v, ones_v, zero_v, acc, sem):
    cid = lax.axis_index("c")
    sid = lax.axis_index("s")
    wid = sid * _NC + cid

    @pl.loop(0, _CH // 16)
    def _fill(i):
        ones_v[pl.ds(i * 16, 16)] = jnp.full((16,), 1.0, jnp.float32)
        zero_v[pl.ds(i * 16, 16)] = jnp.zeros((16,), jnp.float32)

    zb = sid * _SPAN
    for j in range(_NZC):
        pltpu.sync_copy(zero_v, acc.at[pl.ds(zb + j * _CH, _CH)])
    pltpu.sync_copy(zero_v.at[pl.ds(0, _REM)],
                    acc.at[pl.ds(zb + _NZC * _CH, _REM)])
    plsc.subcore_barrier()

    ebase = wid * _EPW

    @pl.loop(0, _NCH)
    def _chunk(c):
        base = pl.multiple_of(ebase + c * _CH, 8)
        pltpu.sync_copy(src_h.at[pl.ds(base, _CH)], src_v)
        pltpu.sync_copy(dst_h.at[pl.ds(base, _CH)], dst_v)

        @pl.loop(0, _CH // 16)
        def _vec(i):
            s = src_v[pl.ds(i * 16, 16)]
            d = dst_v[pl.ds(i * 16, 16)]
            eq = s == d
            srcp_v[pl.ds(i * 16, 16)] = jnp.where(eq, _N, s)
            dstp_v[pl.ds(i * 16, 16)] = jnp.where(eq, _N, d)

        pltpu.sync_copy(ones_v, acc.at[srcp_v], add=True)
        row0 = wid * (_EPW // _RW) + 2 * c
        pltpu.sync_copy(dstp_v.at[pl.ds(0, _RW)], dstp_h.at[row0])
        pltpu.sync_copy(dstp_v.at[pl.ds(_RW, _RW)], dstp_h.at[row0 + 1])

    plsc.subcore_barrier()
    ob = cid * _NPAD + zb
    for j in range(_NZC):
        pltpu.sync_copy(acc.at[pl.ds(zb + j * _CH, _CH)], zero_v)
        pltpu.sync_copy(zero_v, degp_h.at[pl.ds(ob + j * _CH, _CH)])
    pltpu.sync_copy(acc.at[pl.ds(zb + _NZC * _CH, _REM)],
                    zero_v.at[pl.ds(0, _REM)])
    pltpu.sync_copy(zero_v.at[pl.ds(0, _REM)],
                    degp_h.at[pl.ds(ob + _NZC * _CH, _REM)])


@functools.partial(
    pl.kernel,
    out_type=(jax.ShapeDtypeStruct((_NROW, _RW), jnp.int32),
              jax.ShapeDtypeStruct((2 * _NPAD,), jnp.float32)),
    mesh=_sc_mesh,
    scratch_types=[
        pltpu.VMEM((_CH,), jnp.int32),
        pltpu.VMEM((_CH,), jnp.int32),
        pltpu.VMEM((_CH,), jnp.int32),
        pltpu.VMEM((_CH,), jnp.int32),
        pltpu.VMEM((_CH,), jnp.float32),
        pltpu.VMEM((_CH,), jnp.float32),
        pltpu.VMEM_SHARED((_NPAD,), jnp.float32),
        pltpu.SemaphoreType.DMA,
    ],
    compiler_params=pltpu.CompilerParams(use_tc_tiling_on_sc=False),
)
def _deg_prep(*args):
    _deg_prep_body(*args)


def _prop_body(u_h, src_h, dstp_h, s_h,
               sidxA, sidxB, didxA, didxB, rows, acc,
               semI, semG0, semG1, semS0, semS1):
    cid = lax.axis_index("c")
    sid = lax.axis_index("s")
    wid = sid * _NC + cid

    @pl.loop(0, _CH)
    def _z(i):
        rows[i, :] = jnp.zeros((16,), jnp.float32)

    zb = sid * _SPAN
    for j in range(_NZC):
        pltpu.sync_copy(rows, acc.at[pl.ds(zb + j * _CH, _CH)])
    pltpu.sync_copy(rows.at[pl.ds(0, _REM)],
                    acc.at[pl.ds(zb + _NZC * _CH, _REM)])
    plsc.subcore_barrier()

    ebase = wid * _EPW
    rbase = wid * (_EPW // _RW)
    halves = (rows.at[pl.ds(0, _RW)], rows.at[pl.ds(_RW, _RW)])
    semS = (semS0, semS1)
    semG = (semG0, semG1)

    def issue_idx(sec, sb, db):
        base = pl.multiple_of(ebase + sec * _CH, 8)
        pltpu.async_copy(src_h.at[pl.ds(base, _CH)], sb, semI)
        pltpu.async_copy(dstp_h.at[pl.ds(rbase + 2 * sec, 2)], db, semI)

    def wait_idx(sb, db):
        pltpu.make_async_copy(src_h.at[pl.ds(0, _CH)], sb, semI).wait()
        pltpu.make_async_copy(dstp_h.at[pl.ds(0, 2)], db, semI).wait()

    def section(sec, sb, db, other_sb, other_db, first, prefetch):
        if not first:
            for p in range(2):
                pltpu.make_async_copy(u_h.at[pl.ds(0, _RW)], halves[p],
                                      semS[p]).wait()
        wait_idx(sb, db)
        if prefetch is not None:
            prefetch(sec)
        g = []
        for p in range(2):
            g.append(pltpu.async_copy(
                u_h.at[sb.at[pl.ds(p * _RW, _RW)]], halves[p], semG[p]))
        for p in range(2):
            g[p].wait()
            pltpu.async_copy(halves[p], acc.at[db.at[p]], semS[p], add=True)

    issue_idx(0, sidxA, didxA)

    def pf0(sec):
        issue_idx(sec + 1, sidxB, didxB)

    section(0, sidxA, didxA, sidxB, didxB, True, pf0)

    @pl.loop(0, (_NSEC - 1) // 2)
    def _secs(i):
        s0 = 1 + 2 * i
        for b, (sb, db, osb, odb) in enumerate(
                ((sidxB, didxB, sidxA, didxA),
                 (sidxA, didxA, sidxB, didxB))):
            sec = s0 + b

            def pf(s):
                @pl.when(s + 1 < _NSEC)
                def _():
                    pltpu.async_copy(
                        src_h.at[pl.ds(
                            pl.multiple_of(ebase + (s + 1) * _CH, 8), _CH)],
                        osb, semI)
                    pltpu.async_copy(
                        dstp_h.at[pl.ds(rbase + 2 * (s + 1), 2)], odb, semI)

            section(sec, sb, db, osb, odb, False, pf)

    for p in range(2):
        pltpu.make_async_copy(u_h.at[pl.ds(0, _RW)], halves[p],
                              semS[p]).wait()
    plsc.subcore_barrier()
    ob = cid * _NPAD + zb
    for j in range(_NZC):
        pltpu.sync_copy(acc.at[pl.ds(zb + j * _CH, _CH)], rows)
        pltpu.sync_copy(rows, s_h.at[pl.ds(ob + j * _CH, _CH)])
    pltpu.sync_copy(acc.at[pl.ds(zb + _NZC * _CH, _REM)],
                    rows.at[pl.ds(0, _REM)])
    pltpu.sync_copy(rows.at[pl.ds(0, _REM)],
                    s_h.at[pl.ds(ob + _NZC * _CH, _REM)])


@functools.partial(
    pl.kernel,
    out_type=jax.ShapeDtypeStruct((2 * _NPAD, _C), jnp.float32),
    mesh=_sc_mesh,
    scratch_types=[
        pltpu.VMEM((_CH,), jnp.int32),
        pltpu.VMEM((_CH,), jnp.int32),
        pltpu.VMEM((2, _RW), jnp.int32),
        pltpu.VMEM((2, _RW), jnp.int32),
        pltpu.VMEM((_CH, _C), jnp.float32),
        pltpu.VMEM_SHARED((_NPAD, _C), jnp.float32),
        pltpu.SemaphoreType.DMA,
        pltpu.SemaphoreType.DMA,
        pltpu.SemaphoreType.DMA,
        pltpu.SemaphoreType.DMA,
        pltpu.SemaphoreType.DMA,
    ],
    compiler_params=pltpu.CompilerParams(use_tc_tiling_on_sc=False),
)
def _prop(*args):
    _prop_body(*args)


def _e8(dtype=jnp.float32):
    grp = lax.broadcasted_iota(jnp.int32, (8, 128), 1) // 16
    row = lax.broadcasted_iota(jnp.int32, (8, 128), 0)
    return (grp == row).astype(dtype)


def _e8t(dtype=jnp.float32):
    grp = lax.broadcasted_iota(jnp.int32, (128, 8), 0) // 16
    col = lax.broadcasted_iota(jnp.int32, (128, 8), 1)
    return (grp == col).astype(dtype)


_ROWB = pl.BlockSpec((_BP, 128), lambda i: (i, 0))
_S2B = pl.BlockSpec((2, _BP, 128), lambda i: (0, i, 0))


def _mlp_prep_body(xp_ref, w1b_ref, b1b_ref, w2b_ref, b2b_ref, deg_ref,
                   h_ref, disb_ref, d2b_ref, u0_ref):
    h1 = lax.dot_general(xp_ref[...], w1b_ref[...], (((1,), (0,)), ((), ())),
                         preferred_element_type=jnp.float32)
    h1 = jnp.maximum(h1 + b1b_ref[...], 0.0)
    h = lax.dot_general(h1, w2b_ref[...], (((1,), (0,)), ((), ())),
                        preferred_element_type=jnp.float32) + b2b_ref[...]
    deg8 = deg_ref[0] + deg_ref[1]
    dis8 = jnp.where(deg8 > 0.0, lax.rsqrt(deg8), 0.0)
    disb = lax.dot_general(dis8, _e8(), (((1,), (0,)), ((), ())),
                           preferred_element_type=jnp.float32)
    h_ref[...] = h
    disb_ref[...] = disb
    d2b_ref[...] = disb * disb
    u0_ref[...] = disb * h


def _mlp_prep(xp, W1B, b1B, W2B, b2B, deg3):
    return pl.pallas_call(
        _mlp_prep_body,
        grid=(_GRIDP,),
        in_specs=[
            pl.BlockSpec((_BP, 8 * _D), lambda i: (i, 0)),
            pl.BlockSpec((8 * _D, 8 * _H), lambda i: (0, 0)),
            pl.BlockSpec((1, 8 * _H), lambda i: (0, 0)),
            pl.BlockSpec((8 * _H, 128), lambda i: (0, 0)),
            pl.BlockSpec((1, 128), lambda i: (0, 0)),
            pl.BlockSpec((2, _BP, 8), lambda i: (0, i, 0)),
        ],
        out_specs=[_ROWB] * 4,
        out_shape=[jax.ShapeDtypeStruct((_NP8, 128), jnp.float32)] * 4,
    )(xp, W1B, b1B, W2B, b2B, deg3)


def _unext_first_body(s_ref, d2_ref, u_ref):
    u_ref[...] = -d2_ref[...] * (s_ref[0] + s_ref[1])


def _unext_mid_body(s_ref, d2_ref, up_ref, u_ref):
    u_ref[...] = -2.0 * d2_ref[...] * (s_ref[0] + s_ref[1]) - up_ref[...]


def _unext_first(S2, d2b):
    return pl.pallas_call(
        _unext_first_body, grid=(_GRIDP,),
        in_specs=[_S2B, _ROWB], out_specs=_ROWB,
        out_shape=jax.ShapeDtypeStruct((_NP8, 128), jnp.float32),
    )(S2, d2b)


def _unext_mid(S2, d2b, uprev):
    return pl.pallas_call(
        _unext_mid_body, grid=(_GRIDP,),
        in_specs=[_S2B, _ROWB, _ROWB], out_specs=_ROWB,
        out_shape=jax.ShapeDtypeStruct((_NP8, 128), jnp.float32),
    )(S2, d2b, uprev)


def _final_body(*refs):
    s_refs = refs[:_K]
    h_ref, disb_ref, wp_ref, bp_ref, g_ref, out_ref = refs[_K:]
    e8 = _e8()
    e8t = _e8t()
    wp = wp_ref[...]
    bp = bp_ref[0, 0]
    disb = disb_ref[...]

    def term(tx, k):
        r8 = lax.dot_general(tx * wp, e8t, (((1,), (0,)), ((), ())),
                             preferred_element_type=jnp.float32) + bp
        e = g_ref[0, k] / (1.0 + jnp.exp(-r8))
        eta = lax.dot_general(e, e8, (((1,), (0,)), ((), ())),
                              preferred_element_type=jnp.float32)
        return eta * tx

    tx_a = h_ref[...]
    acc = term(tx_a, 0)
    tx_b = -disb * (s_refs[0][0] + s_refs[0][1])
    acc += term(tx_b, 1)
    for k in range(2, _K + 1):
        s = s_refs[k - 1][0] + s_refs[k - 1][1]
        tx_n = -2.0 * disb * s - tx_a
        acc += term(tx_n, k)
        tx_a, tx_b = tx_b, tx_n

    m = jnp.max(acc, axis=1, keepdims=True)
    z = acc - m
    s8 = lax.dot_general(jnp.exp(z), e8t, (((1,), (0,)), ((), ())),
                         preferred_element_type=jnp.float32)
    l128 = lax.dot_general(jnp.log(s8), e8, (((1,), (0,)), ((), ())),
                           preferred_element_type=jnp.float32)
    out_ref[...] = z - l128


def _final(Sp, h, disb, wp128, bpr, gamma):
    return pl.pallas_call(
        _final_body,
        grid=(_GRIDP,),
        in_specs=[_S2B] * _K + [
            _ROWB, _ROWB,
            pl.BlockSpec((1, 128), lambda i: (0, 0)),
            pl.BlockSpec(memory_space=pltpu.SMEM),
            pl.BlockSpec(memory_space=pltpu.SMEM),
        ],
        out_specs=_ROWB,
        out_shape=jax.ShapeDtypeStruct((_NP8, 128), jnp.float32),
    )(*Sp, h, disb, wp128, bpr, gamma)


def kernel(x, edge_index, W1, b1, W2, b2, gamma, W_proj, b_proj):
    src = edge_index[0]
    dst = edge_index[1]
    eye8 = jnp.eye(8, dtype=jnp.float32)
    W1B = jnp.kron(eye8, W1.T)
    W2B = jnp.kron(eye8, W2.T)
    b1B = jnp.tile(b1.reshape(1, _H), (1, 8))
    b2B = jnp.tile(b2.reshape(1, _C), (1, 8))
    wp128 = jnp.tile(W_proj.reshape(1, _C), (1, 8))
    bpr = b_proj.reshape(1, 1)
    xp = x.reshape(_NP8, 8 * _D)

    dstp, degp = _deg_prep(src, dst)
    deg3 = degp.reshape(2, _NPAD)[:, :_N].reshape(2, _NP8, 8)
    h, disb, d2b, u0 = _mlp_prep(xp, W1B, b1B, W2B, b2B, deg3)

    Sp = []
    upk = u0
    S0 = _prop(u0.reshape(_N, _C), src, dstp).reshape(2, _SROW1, 128)
    Sp.append(S0)
    uk = _unext_first(S0, d2b)
    for k in range(1, _K):
        Sk = _prop(uk.reshape(_N, _C), src, dstp).reshape(2, _SROW1, 128)
        Sp.append(Sk)
        if k < _K - 1:
            uk, upk = _unext_mid(Sk, d2b, upk), uk

    out = _final(Sp, h, disb, wp128, bpr, gamma)
    return out.reshape(_N, _C)
